# baseline (speedup 1.0000x reference)
import jax
import jax.numpy as jnp
from jax.experimental import pallas as pl

SIGMA_ = 5.0
K_ = 5
BINS_ = 313


IDX_MASK_ = (1 << 9) - 1
INT_MAX_ = jnp.iinfo(jnp.int32).max


def _ct2_tile_kernel(qx_ref, qy_ref, qsq_ref, qi_ref, pts_ref, out_ref):
    bf = jnp.bfloat16
    f32 = jnp.float32
    px = pts_ref[0, 0]
    py = pts_ref[0, 1]
    pxb = px.astype(bf).astype(f32)
    pyb = py.astype(bf).astype(f32)
    p_sq = px * px + py * py

    qxb = qx_ref[...].astype(f32)
    qyb = qy_ref[...].astype(f32)
    q_sq = qsq_ref[...]
    qi = qi_ref[...]

    cross = qxb * pxb[None] + qyb * pyb[None]
    d2 = jnp.maximum((q_sq + p_sq[None]) - 2.0 * cross, 0.0)
    key0 = (jax.lax.bitcast_convert_type(d2, jnp.int32) & ~IDX_MASK_) | qi

    cur = key0
    kms = []
    for k in range(K_):
        km = jnp.min(cur, axis=0)
        kms.append(km)
        if k + 1 < K_:
            cur = jnp.where(cur == km[None], INT_MAX_, cur)

    scale = jnp.float32(-1.0 / (2.0 * SIGMA_ * SIGMA_))
    ws = [jnp.exp(jax.lax.bitcast_convert_type(km & ~IDX_MASK_, f32) * scale)
          for km in kms]
    nlogs = -jnp.log(ws[0] + ws[1] + ws[2] + ws[3] + ws[4])

    e = jnp.exp(d2 * scale + nlogs[None])
    out_ref[0] = jnp.where(key0 <= kms[K_ - 1][None], e, 0.0)


@jax.jit
def kernel(gt_ab, q_ab):
    bs, _, H, W = gt_ab.shape
    hb = 8
    bf = jnp.bfloat16
    qxb = q_ab[:, 0].astype(bf).reshape(BINS_, 1, 1)
    qyb = q_ab[:, 1].astype(bf).reshape(BINS_, 1, 1)
    qsq = (q_ab[:, 0] * q_ab[:, 0] + q_ab[:, 1] * q_ab[:, 1]).reshape(BINS_, 1, 1)
    qi = jnp.arange(BINS_, dtype=jnp.int32).reshape(BINS_, 1, 1)
    grid = (bs, H // hb)
    qspec = pl.BlockSpec((BINS_, 1, 1), lambda i, j: (0, 0, 0))
    return pl.pallas_call(
        _ct2_tile_kernel,
        grid=grid,
        in_specs=[
            qspec, qspec, qspec, qspec,
            pl.BlockSpec((1, 2, hb, W), lambda i, j: (i, 0, j, 0)),
        ],
        out_specs=pl.BlockSpec((1, BINS_, hb, W), lambda i, j: (i, 0, j, 0)),
        out_shape=jax.ShapeDtypeStruct((bs, BINS_, H, W), jnp.float32),
    )(qxb, qyb, qsq, qi, gt_ab)


# speedup vs baseline: 1.1089x; 1.1089x over previous
import jax
import jax.numpy as jnp
from jax.experimental import pallas as pl

SIGMA_ = 5.0
K_ = 5
BINS_ = 313


IDX_MASK_ = (1 << 9) - 1
INT_MAX_ = jnp.iinfo(jnp.int32).max


def _ct2_tile_kernel(qx_ref, qy_ref, qsq_ref, qi_ref, pts_ref, out_ref):
    bf = jnp.bfloat16
    f32 = jnp.float32
    px = pts_ref[0, 0]
    py = pts_ref[0, 1]
    pxb = px.astype(bf).astype(f32)
    pyb = py.astype(bf).astype(f32)
    p_sq = px * px + py * py

    qxb = qx_ref[...].astype(f32)
    qyb = qy_ref[...].astype(f32)
    q_sq = qsq_ref[...]
    qi = qi_ref[...]

    cross = qxb * pxb[None] + qyb * pyb[None]
    d2 = jnp.maximum((q_sq + p_sq[None]) - 2.0 * cross, 0.0)
    key0 = (jax.lax.bitcast_convert_type(d2, jnp.int32) & ~IDX_MASK_) | qi

    cur = key0
    kms = []
    for k in range(K_):
        km = jnp.min(cur, axis=0)
        kms.append(km)
        if k + 1 < K_:
            cur = jnp.where(cur == km[None], INT_MAX_, cur)

    scale = jnp.float32(-1.0 / (2.0 * SIGMA_ * SIGMA_))
    ws = [jnp.exp(jax.lax.bitcast_convert_type(km & ~IDX_MASK_, f32) * scale)
          for km in kms]
    nlogs = -jnp.log(ws[0] + ws[1] + ws[2] + ws[3] + ws[4])

    e = jnp.exp(d2 * scale + nlogs[None])
    out_ref[0] = jnp.where(key0 <= kms[K_ - 1][None], e, 0.0)


@jax.jit
def kernel(gt_ab, q_ab):
    bs, _, H, W = gt_ab.shape
    hb = 32
    bf = jnp.bfloat16
    qxb = q_ab[:, 0].astype(bf).reshape(BINS_, 1, 1)
    qyb = q_ab[:, 1].astype(bf).reshape(BINS_, 1, 1)
    qsq = (q_ab[:, 0] * q_ab[:, 0] + q_ab[:, 1] * q_ab[:, 1]).reshape(BINS_, 1, 1)
    qi = jnp.arange(BINS_, dtype=jnp.int32).reshape(BINS_, 1, 1)
    grid = (bs, H // hb)
    qspec = pl.BlockSpec((BINS_, 1, 1), lambda i, j: (0, 0, 0))
    return pl.pallas_call(
        _ct2_tile_kernel,
        grid=grid,
        in_specs=[
            qspec, qspec, qspec, qspec,
            pl.BlockSpec((1, 2, hb, W), lambda i, j: (i, 0, j, 0)),
        ],
        out_specs=pl.BlockSpec((1, BINS_, hb, W), lambda i, j: (i, 0, j, 0)),
        out_shape=jax.ShapeDtypeStruct((bs, BINS_, H, W), jnp.float32),
    )(qxb, qyb, qsq, qi, gt_ab)


# speedup vs baseline: 1.2515x; 1.1286x over previous
import jax
import jax.numpy as jnp
from jax.experimental import pallas as pl

SIGMA_ = 5.0
K_ = 5
BINS_ = 313


IDX_MASK_ = (1 << 9) - 1
INT_MAX_ = jnp.iinfo(jnp.int32).max
BIAS_ = 1 << 23


def _ct2_tile_kernel(qx_ref, qy_ref, qsq_ref, qi_ref, pts_ref, out_ref):
    bf = jnp.bfloat16
    f32 = jnp.float32
    px = pts_ref[0, 0]
    py = pts_ref[0, 1]
    pxb = px.astype(bf).astype(f32)
    pyb = py.astype(bf).astype(f32)
    p_sq = px * px + py * py

    qxb = qx_ref[...].astype(f32)
    qyb = qy_ref[...].astype(f32)
    q_sq = qsq_ref[...]
    qi = qi_ref[...]

    cross = qxb * pxb[None] + qyb * pyb[None]
    d2 = jnp.maximum((q_sq + p_sq[None]) - 2.0 * cross, 0.0)
    ki = (jax.lax.bitcast_convert_type(d2, jnp.int32) & ~IDX_MASK_) | qi
    key0 = jax.lax.bitcast_convert_type(ki + BIAS_, f32)

    cur = key0
    kms = []
    inf = jnp.float32(jnp.inf)
    for k in range(K_):
        km = jnp.min(cur, axis=0)
        kms.append(km)
        if k + 1 < K_:
            cur = jnp.where(cur == km[None], inf, cur)

    scale = jnp.float32(-1.0 / (2.0 * SIGMA_ * SIGMA_))
    ws = [jnp.exp(jax.lax.bitcast_convert_type(
              (jax.lax.bitcast_convert_type(km, jnp.int32) - BIAS_)
              & ~IDX_MASK_, f32) * scale)
          for km in kms]
    nlogs = -jnp.log(ws[0] + ws[1] + ws[2] + ws[3] + ws[4])

    e = jnp.exp(d2 * scale + nlogs[None])
    out_ref[0] = jnp.where(key0 <= kms[K_ - 1][None], e, 0.0)


@jax.jit
def kernel(gt_ab, q_ab):
    bs, _, H, W = gt_ab.shape
    hb = 16
    bf = jnp.bfloat16
    qxb = q_ab[:, 0].astype(bf).reshape(BINS_, 1, 1)
    qyb = q_ab[:, 1].astype(bf).reshape(BINS_, 1, 1)
    qsq = (q_ab[:, 0] * q_ab[:, 0] + q_ab[:, 1] * q_ab[:, 1]).reshape(BINS_, 1, 1)
    qi = jnp.arange(BINS_, dtype=jnp.int32).reshape(BINS_, 1, 1)
    grid = (bs, H // hb)
    qspec = pl.BlockSpec((BINS_, 1, 1), lambda i, j: (0, 0, 0))
    return pl.pallas_call(
        _ct2_tile_kernel,
        grid=grid,
        in_specs=[
            qspec, qspec, qspec, qspec,
            pl.BlockSpec((1, 2, hb, W), lambda i, j: (i, 0, j, 0)),
        ],
        out_specs=pl.BlockSpec((1, BINS_, hb, W), lambda i, j: (i, 0, j, 0)),
        out_shape=jax.ShapeDtypeStruct((bs, BINS_, H, W), jnp.float32),
    )(qxb, qyb, qsq, qi, gt_ab)
